# Initial kernel scaffold; baseline (speedup 1.0000x reference)
import functools

import jax
import jax.numpy as jnp
from jax import lax
from jax.experimental import pallas as pl
from jax.experimental.pallas import tpu as pltpu
from jax.experimental.pallas import tpu_sc as plsc

N = 6_400_000
B = 100_000
NC = 2
NS = 16
NW = NC * NS
C = N // NW
T = 10_000
SLICE = 3_128
BP = NW * SLICE
L = 16

_MESH = plsc.VectorSubcoreMesh(core_axis_name="c", subcore_axis_name="s")


def _wid():
    return lax.axis_index("s") * NC + lax.axis_index("c")


def _k1_body(x_hbm, b_hbm, ps_hbm, acc, bbuf, vbuf):
    w = _wid()

    def zero(i, _):
        acc[pl.ds(i * L, L)] = jnp.zeros((L,), jnp.float32)
        return 0

    lax.fori_loop(0, BP // L, zero, 0, unroll=8)

    iota = lax.iota(jnp.int32, L)
    idx_l = jnp.minimum(iota + 1, L - 1)
    idx_r = jnp.maximum(iota - 1, 0)
    last = iota == (L - 1)
    first = iota == 0

    def tile(t, _):
        base = w * C + t * T
        pltpu.sync_copy(b_hbm.at[pl.ds(base, T)], bbuf)
        pltpu.sync_copy(x_hbm.at[pl.ds(base, T)], vbuf)

        def vreg(k, _):
            b = bbuf[pl.ds(k * L, L)]
            v = vbuf[pl.ds(k * L, L)]
            s = plsc.cumsum(v)
            b_next = jnp.take(b, idx_l, mode="promise_in_bounds")
            b_prev = jnp.take(b, idx_r, mode="promise_in_bounds")
            is_end = (b != b_next) | last
            is_start = (b != b_prev) | first
            plsc.addupdate_scatter(acc, [b], s, mask=is_end)
            plsc.addupdate_scatter(acc, [b], v - s, mask=is_start)
            return 0

        lax.fori_loop(0, T // L, vreg, 0)
        return 0

    lax.fori_loop(0, C // T, tile, 0)
    pltpu.sync_copy(acc, ps_hbm.at[w])


def _k2_body(ps_hbm, tc_hbm, na_hbm, lo_hbm, accs, tbuf):
    w = _wid()
    off = w * SLICE
    pltpu.sync_copy(tc_hbm.at[pl.ds(off, SLICE)], accs)

    def row(v, _):
        pltpu.sync_copy(ps_hbm.at[v, pl.ds(off, SLICE)], tbuf)

        def sub(k, _):
            sl = pl.ds(k * L, L)
            accs[sl] = accs[sl] - tbuf[sl]
            return 0

        lax.fori_loop(0, SLICE // L, sub, 0)
        return 0

    lax.fori_loop(0, NW, row, 0)

    pltpu.sync_copy(na_hbm.at[pl.ds(off, SLICE)], tbuf)

    def div(k, _):
        sl = pl.ds(k * L, L)
        accs[sl] = accs[sl] / tbuf[sl]
        return 0

    lax.fori_loop(0, SLICE // L, div, 0)
    pltpu.sync_copy(accs, lo_hbm.at[pl.ds(off, SLICE)])


def _k3_body(x_hbm, b_hbm, lo_hbm, out_hbm, lt, bbuf, vbuf, obuf):
    w = _wid()
    pltpu.sync_copy(lo_hbm, lt)

    def tile(t, _):
        base = w * C + t * T
        pltpu.sync_copy(b_hbm.at[pl.ds(base, T)], bbuf)
        pltpu.sync_copy(x_hbm.at[pl.ds(base, T)], vbuf)

        def vreg(k, _):
            b = bbuf[pl.ds(k * L, L)]
            v = vbuf[pl.ds(k * L, L)]
            g = plsc.load_gather(lt, [b])
            obuf[pl.ds(k * L, L)] = v + g
            return 0

        lax.fori_loop(0, T // L, vreg, 0)
        pltpu.sync_copy(obuf, out_hbm.at[pl.ds(base, T)])
        return 0

    lax.fori_loop(0, C // T, tile, 0)


_k1 = functools.partial(
    pl.kernel,
    out_type=jax.ShapeDtypeStruct((NW, BP), jnp.float32),
    mesh=_MESH,
    scratch_types=[
        pltpu.VMEM((BP,), jnp.float32),
        pltpu.VMEM((T,), jnp.int32),
        pltpu.VMEM((T,), jnp.float32),
    ],
)(_k1_body)

_k2 = functools.partial(
    pl.kernel,
    out_type=jax.ShapeDtypeStruct((BP,), jnp.float32),
    mesh=_MESH,
    scratch_types=[
        pltpu.VMEM((SLICE,), jnp.float32),
        pltpu.VMEM((SLICE,), jnp.float32),
    ],
)(_k2_body)

_k3 = functools.partial(
    pl.kernel,
    out_type=jax.ShapeDtypeStruct((N,), jnp.float32),
    mesh=_MESH,
    scratch_types=[
        pltpu.VMEM((BP,), jnp.float32),
        pltpu.VMEM((T,), jnp.int32),
        pltpu.VMEM((T,), jnp.float32),
        pltpu.VMEM((T,), jnp.float32),
    ],
)(_k3_body)


def kernel(node_outputs, batch, total_charge, n_atoms):
    x = node_outputs.reshape(N)
    tc = jnp.concatenate([total_charge, jnp.zeros((BP - B,), jnp.float32)])
    na = jnp.concatenate([n_atoms, jnp.ones((BP - B,), jnp.float32)])
    ps = _k1(x, batch)
    leftover = _k2(ps, tc, na)
    return _k3(x, batch, leftover)


# speedup vs baseline: 171.8647x; 171.8647x over previous
import functools

import jax
import jax.numpy as jnp
from jax import lax
from jax.experimental import pallas as pl
from jax.experimental.pallas import tpu as pltpu
from jax.experimental.pallas import tpu_sc as plsc

N = 6_400_000
B = 100_000
NC = 2
NS = 16
NW = NC * NS
C = N // NW
T = 10_000
SLICE = 3_136
BP = NW * SLICE
L = 16

_MESH = plsc.VectorSubcoreMesh(core_axis_name="c", subcore_axis_name="s")
_PARAMS = pltpu.CompilerParams(needs_layout_passes=False)


def _wid():
    return lax.axis_index("s") * NC + lax.axis_index("c")


def _k1_body(x_hbm, b_hbm, ps_hbm, acc, bbuf, vbuf):
    w = _wid()

    def zero(i, _):
        acc[pl.ds(i * L, L)] = jnp.zeros((L,), jnp.float32)
        return 0

    lax.fori_loop(0, BP // L, zero, 0, unroll=8)

    iota = lax.iota(jnp.int32, L)
    idx_l = jnp.minimum(iota + 1, L - 1)
    idx_r = jnp.maximum(iota - 1, 0)
    last = iota == (L - 1)
    first = iota == 0

    def tile(t, _):
        base = pl.multiple_of(w * C + t * T, 8)
        pltpu.sync_copy(b_hbm.at[pl.ds(base, T)], bbuf)
        pltpu.sync_copy(x_hbm.at[pl.ds(base, T)], vbuf)

        def vreg(k, _):
            b = bbuf[pl.ds(k * L, L)]
            v = vbuf[pl.ds(k * L, L)]
            s = plsc.cumsum(v)
            b_next = jnp.take_along_axis(b, idx_l, axis=0)
            b_prev = jnp.take_along_axis(b, idx_r, axis=0)
            is_end = (b != b_next) | last
            is_start = (b != b_prev) | first
            plsc.addupdate_scatter(acc, [b], s, mask=is_end)
            plsc.addupdate_scatter(acc, [b], v - s, mask=is_start)
            return 0

        lax.fori_loop(0, T // L, vreg, 0)
        return 0

    lax.fori_loop(0, C // T, tile, 0)
    pltpu.sync_copy(acc, ps_hbm.at[pl.ds(pl.multiple_of(w * BP, 8), BP)])


def _k2_body(ps_hbm, tc_hbm, na_hbm, lo_hbm, accs, tbuf):
    w = _wid()
    off = pl.multiple_of(w * SLICE, 8)
    pltpu.sync_copy(tc_hbm.at[pl.ds(off, SLICE)], accs)

    def row(v, _):
        pltpu.sync_copy(ps_hbm.at[pl.ds(pl.multiple_of(v * BP + off, 8), SLICE)], tbuf)

        def sub(k, _):
            sl = pl.ds(k * L, L)
            accs[sl] = accs[sl] - tbuf[sl]
            return 0

        lax.fori_loop(0, SLICE // L, sub, 0)
        return 0

    lax.fori_loop(0, NW, row, 0)

    pltpu.sync_copy(na_hbm.at[pl.ds(off, SLICE)], tbuf)

    def div(k, _):
        sl = pl.ds(k * L, L)
        accs[sl] = accs[sl] / tbuf[sl]
        return 0

    lax.fori_loop(0, SLICE // L, div, 0)
    pltpu.sync_copy(accs, lo_hbm.at[pl.ds(off, SLICE)])


def _k3_body(x_hbm, b_hbm, lo_hbm, out_hbm, lt, bbuf, vbuf, obuf):
    w = _wid()
    pltpu.sync_copy(lo_hbm, lt)

    def tile(t, _):
        base = pl.multiple_of(w * C + t * T, 8)
        pltpu.sync_copy(b_hbm.at[pl.ds(base, T)], bbuf)
        pltpu.sync_copy(x_hbm.at[pl.ds(base, T)], vbuf)

        def vreg(k, _):
            b = bbuf[pl.ds(k * L, L)]
            v = vbuf[pl.ds(k * L, L)]
            g = plsc.load_gather(lt, [b])
            obuf[pl.ds(k * L, L)] = v + g
            return 0

        lax.fori_loop(0, T // L, vreg, 0)
        pltpu.sync_copy(obuf, out_hbm.at[pl.ds(base, T)])
        return 0

    lax.fori_loop(0, C // T, tile, 0)


_k1 = functools.partial(
    pl.kernel,
    out_type=jax.ShapeDtypeStruct((NW * BP,), jnp.float32),
    mesh=_MESH,
    compiler_params=_PARAMS,
    scratch_types=[
        pltpu.VMEM((BP,), jnp.float32),
        pltpu.VMEM((T,), jnp.int32),
        pltpu.VMEM((T,), jnp.float32),
    ],
)(_k1_body)

_k2 = functools.partial(
    pl.kernel,
    out_type=jax.ShapeDtypeStruct((BP,), jnp.float32),
    mesh=_MESH,
    compiler_params=_PARAMS,
    scratch_types=[
        pltpu.VMEM((SLICE,), jnp.float32),
        pltpu.VMEM((SLICE,), jnp.float32),
    ],
)(_k2_body)

_k3 = functools.partial(
    pl.kernel,
    out_type=jax.ShapeDtypeStruct((N,), jnp.float32),
    mesh=_MESH,
    compiler_params=_PARAMS,
    scratch_types=[
        pltpu.VMEM((BP,), jnp.float32),
        pltpu.VMEM((T,), jnp.int32),
        pltpu.VMEM((T,), jnp.float32),
        pltpu.VMEM((T,), jnp.float32),
    ],
)(_k3_body)


def kernel(node_outputs, batch, total_charge, n_atoms):
    x = node_outputs.reshape(N)
    tc = jnp.concatenate([total_charge, jnp.zeros((BP - B,), jnp.float32)])
    na = jnp.concatenate([n_atoms, jnp.ones((BP - B,), jnp.float32)])
    ps = _k1(x, batch)
    leftover = _k2(ps, tc, na)
    return _k3(x, batch, leftover)


# speedup vs baseline: 207.9241x; 1.2098x over previous
import functools

import jax
import jax.numpy as jnp
from jax import lax
from jax.experimental import pallas as pl
from jax.experimental.pallas import tpu as pltpu
from jax.experimental.pallas import tpu_sc as plsc

N = 6_400_000
B = 100_000
NC = 2
NS = 16
NW = NC * NS
C = N // NW
T = 4_000
NT = C // T
SLICE = 3_136
BP = NW * SLICE
L = 16
H = 16

_MESH = plsc.VectorSubcoreMesh(core_axis_name="c", subcore_axis_name="s")
_PARAMS = pltpu.CompilerParams(needs_layout_passes=False)


def _wid():
    return lax.axis_index("s") * NC + lax.axis_index("c")


def _k1_body(x_hbm, b_hbm, ps_hbm, acc, bb0, bb1, vb0, vb1, bs0, bs1, vs0, vs1):
    w = _wid()
    bbs, vbs = (bb0, bb1), (vb0, vb1)
    bsem, vsem = (bs0, bs1), (vs0, vs1)

    def zero(i, _):
        acc[pl.ds(i * L, L)] = jnp.zeros((L,), jnp.float32)
        return 0

    lax.fori_loop(0, BP // L, zero, 0, unroll=8)

    def base(t):
        return pl.multiple_of(w * C + t * T, 8)

    def issue(t, p):
        pltpu.async_copy(b_hbm.at[pl.ds(base(t), T)], bbs[p].at[pl.ds(H, T)],
                         bsem[p])
        pltpu.async_copy(x_hbm.at[pl.ds(base(t), T)], vbs[p], vsem[p])

    issue(0, 0)

    iota = lax.iota(jnp.int32, L)
    last = iota == (L - 1)
    first = iota == 0

    def compute(bb, vb):
        def vreg(k, _):
            off = H + k * L
            b = bb[pl.ds(off, L)]
            v = vb[pl.ds(k * L, L)]
            s = plsc.cumsum(v)
            b_next = bb[pl.ds(off + 1, L)]
            b_prev = bb[pl.ds(off - 1, L)]
            is_end = (b != b_next) | last
            is_start = (b != b_prev) | first
            plsc.addupdate_scatter(acc, [b], s, mask=is_end)
            plsc.addupdate_scatter(acc, [b], v - s, mask=is_start)
            return 0

        lax.fori_loop(0, T // L, vreg, 0, unroll=4)

    def outer(to, _):
        for p in (0, 1):
            t = 2 * to + p

            @pl.when(t + 1 < NT)
            def _():
                issue(t + 1, 1 - p)

            pltpu.make_async_copy(b_hbm.at[pl.ds(base(t), T)],
                                  bbs[p].at[pl.ds(H, T)], bsem[p]).wait()
            pltpu.make_async_copy(x_hbm.at[pl.ds(base(t), T)], vbs[p],
                                  vsem[p]).wait()
            compute(bbs[p], vbs[p])
        return 0

    lax.fori_loop(0, NT // 2, outer, 0)
    pltpu.sync_copy(acc, ps_hbm.at[pl.ds(pl.multiple_of(w * BP, 8), BP)])


def _k2_body(ps_hbm, tc_hbm, na_hbm, lo_hbm, accs, tb0, tb1, ts0, ts1):
    w = _wid()
    off = pl.multiple_of(w * SLICE, 8)
    tbs, tsem = (tb0, tb1), (ts0, ts1)
    pltpu.sync_copy(tc_hbm.at[pl.ds(off, SLICE)], accs)

    def issue(v, p):
        pltpu.async_copy(
            ps_hbm.at[pl.ds(pl.multiple_of(v * BP + off, 8), SLICE)],
            tbs[p], tsem[p])

    issue(0, 0)

    def outer(vo, _):
        for p in (0, 1):
            v = 2 * vo + p

            @pl.when(v + 1 < NW)
            def _():
                issue(v + 1, 1 - p)

            pltpu.make_async_copy(
                ps_hbm.at[pl.ds(pl.multiple_of(v * BP + off, 8), SLICE)],
                tbs[p], tsem[p]).wait()

            def sub(k, _):
                sl = pl.ds(k * L, L)
                accs[sl] = accs[sl] - tbs[p][sl]
                return 0

            lax.fori_loop(0, SLICE // L, sub, 0, unroll=4)
        return 0

    lax.fori_loop(0, NW // 2, outer, 0)

    pltpu.sync_copy(na_hbm.at[pl.ds(off, SLICE)], tb0)

    def div(k, _):
        sl = pl.ds(k * L, L)
        accs[sl] = accs[sl] / tb0[sl]
        return 0

    lax.fori_loop(0, SLICE // L, div, 0, unroll=4)
    pltpu.sync_copy(accs, lo_hbm.at[pl.ds(off, SLICE)])


def _k3_body(x_hbm, b_hbm, lo_hbm, out_hbm, lt, bb0, bb1, vb0, vb1, ob0, ob1,
             lsem, bs0, bs1, vs0, vs1, os0, os1):
    w = _wid()
    bbs, vbs, obs = (bb0, bb1), (vb0, vb1), (ob0, ob1)
    bsem, vsem, osem = (bs0, bs1), (vs0, vs1), (os0, os1)

    pltpu.async_copy(lo_hbm, lt, lsem)

    def base(t):
        return pl.multiple_of(w * C + t * T, 8)

    def issue(t, p):
        pltpu.async_copy(b_hbm.at[pl.ds(base(t), T)], bbs[p], bsem[p])
        pltpu.async_copy(x_hbm.at[pl.ds(base(t), T)], vbs[p], vsem[p])

    issue(0, 0)
    pltpu.make_async_copy(lo_hbm, lt, lsem).wait()

    def outer(to, _):
        for p in (0, 1):
            t = 2 * to + p

            @pl.when(t + 1 < NT)
            def _():
                issue(t + 1, 1 - p)

            pltpu.make_async_copy(b_hbm.at[pl.ds(base(t), T)], bbs[p],
                                  bsem[p]).wait()
            pltpu.make_async_copy(x_hbm.at[pl.ds(base(t), T)], vbs[p],
                                  vsem[p]).wait()

            @pl.when(t >= 2)
            def _():
                pltpu.make_async_copy(obs[p], out_hbm.at[pl.ds(base(t - 2), T)],
                                      osem[p]).wait()

            def vreg(k, _):
                sl = pl.ds(k * L, L)
                b = bbs[p][sl]
                v = vbs[p][sl]
                g = plsc.load_gather(lt, [b])
                obs[p][sl] = v + g
                return 0

            lax.fori_loop(0, T // L, vreg, 0, unroll=4)
            pltpu.async_copy(obs[p], out_hbm.at[pl.ds(base(t), T)], osem[p])
        return 0

    lax.fori_loop(0, NT // 2, outer, 0)
    pltpu.make_async_copy(ob0, out_hbm.at[pl.ds(base(NT - 2), T)], os0).wait()
    pltpu.make_async_copy(ob1, out_hbm.at[pl.ds(base(NT - 1), T)], os1).wait()


_k1 = functools.partial(
    pl.kernel,
    out_type=jax.ShapeDtypeStruct((NW * BP,), jnp.float32),
    mesh=_MESH,
    compiler_params=_PARAMS,
    scratch_types=[
        pltpu.VMEM((BP,), jnp.float32),
        pltpu.VMEM((T + 2 * H,), jnp.int32),
        pltpu.VMEM((T + 2 * H,), jnp.int32),
        pltpu.VMEM((T,), jnp.float32),
        pltpu.VMEM((T,), jnp.float32),
        pltpu.SemaphoreType.DMA,
        pltpu.SemaphoreType.DMA,
        pltpu.SemaphoreType.DMA,
        pltpu.SemaphoreType.DMA,
    ],
)(_k1_body)

_k2 = functools.partial(
    pl.kernel,
    out_type=jax.ShapeDtypeStruct((BP,), jnp.float32),
    mesh=_MESH,
    compiler_params=_PARAMS,
    scratch_types=[
        pltpu.VMEM((SLICE,), jnp.float32),
        pltpu.VMEM((SLICE,), jnp.float32),
        pltpu.VMEM((SLICE,), jnp.float32),
        pltpu.SemaphoreType.DMA,
        pltpu.SemaphoreType.DMA,
    ],
)(_k2_body)

_k3 = functools.partial(
    pl.kernel,
    out_type=jax.ShapeDtypeStruct((N,), jnp.float32),
    mesh=_MESH,
    compiler_params=_PARAMS,
    scratch_types=[
        pltpu.VMEM((BP,), jnp.float32),
        pltpu.VMEM((T,), jnp.int32),
        pltpu.VMEM((T,), jnp.int32),
        pltpu.VMEM((T,), jnp.float32),
        pltpu.VMEM((T,), jnp.float32),
        pltpu.VMEM((T,), jnp.float32),
        pltpu.VMEM((T,), jnp.float32),
        pltpu.SemaphoreType.DMA,
        pltpu.SemaphoreType.DMA,
        pltpu.SemaphoreType.DMA,
        pltpu.SemaphoreType.DMA,
        pltpu.SemaphoreType.DMA,
        pltpu.SemaphoreType.DMA,
        pltpu.SemaphoreType.DMA,
    ],
)(_k3_body)


def kernel(node_outputs, batch, total_charge, n_atoms):
    x = node_outputs.reshape(N)
    tc = jnp.concatenate([total_charge, jnp.zeros((BP - B,), jnp.float32)])
    na = jnp.concatenate([n_atoms, jnp.ones((BP - B,), jnp.float32)])
    ps = _k1(x, batch)
    leftover = _k2(ps, tc, na)
    return _k3(x, batch, leftover)


# speedup vs baseline: 460.5963x; 2.2152x over previous
import functools

import jax
import jax.numpy as jnp
from jax import lax
from jax.experimental import pallas as pl
from jax.experimental.pallas import tpu as pltpu
from jax.experimental.pallas import tpu_sc as plsc

N = 6_400_000
B = 100_000
NC = 2
NS = 16
NW = NC * NS
C = N // NW
T = 4_000
NT = C // T
SLICE = 3_136
BP = NW * SLICE
L = 16
H = 16

_MESH = plsc.VectorSubcoreMesh(core_axis_name="c", subcore_axis_name="s")
_PARAMS = pltpu.CompilerParams(needs_layout_passes=False)


def _wid():
    return lax.axis_index("s") * NC + lax.axis_index("c")


def _k1_body(x_hbm, b_hbm, ps_hbm, acc, bb0, bb1, vb0, vb1, bs0, bs1, vs0, vs1):
    w = _wid()
    bbs, vbs = (bb0, bb1), (vb0, vb1)
    bsem, vsem = (bs0, bs1), (vs0, vs1)

    @plsc.parallel_loop(0, BP // L, unroll=8)
    def zero(i):
        acc[pl.ds(i * L, L)] = jnp.zeros((L,), jnp.float32)

    def base(t):
        return pl.multiple_of(w * C + t * T, 8)

    def issue(t, p):
        pltpu.async_copy(b_hbm.at[pl.ds(base(t), T)], bbs[p].at[pl.ds(H, T)],
                         bsem[p])
        pltpu.async_copy(x_hbm.at[pl.ds(base(t), T)], vbs[p], vsem[p])

    issue(0, 0)

    iota = lax.iota(jnp.int32, L)
    last = iota == (L - 1)
    first = iota == 0

    def compute(bb, vb):
        @plsc.parallel_loop(0, T // L, unroll=4)
        def vreg(k):
            off = H + k * L
            b = bb[pl.ds(off, L)]
            v = vb[pl.ds(k * L, L)]
            s = plsc.cumsum(v)
            b_next = bb[pl.ds(off + 1, L)]
            b_prev = bb[pl.ds(off - 1, L)]
            is_end = (b != b_next) | last
            is_start = (b != b_prev) | first
            plsc.addupdate_scatter(acc, [b], s, mask=is_end)
            plsc.addupdate_scatter(acc, [b], v - s, mask=is_start)

    def outer(to, _):
        for p in (0, 1):
            t = 2 * to + p

            @pl.when(t + 1 < NT)
            def _():
                issue(t + 1, 1 - p)

            pltpu.make_async_copy(b_hbm.at[pl.ds(base(t), T)],
                                  bbs[p].at[pl.ds(H, T)], bsem[p]).wait()
            pltpu.make_async_copy(x_hbm.at[pl.ds(base(t), T)], vbs[p],
                                  vsem[p]).wait()
            compute(bbs[p], vbs[p])
        return 0

    lax.fori_loop(0, NT // 2, outer, 0)
    pltpu.sync_copy(acc, ps_hbm.at[pl.ds(pl.multiple_of(w * BP, 8), BP)])


def _k2_body(ps_hbm, tc_hbm, na_hbm, lo_hbm, accs, tb0, tb1, ts0, ts1):
    w = _wid()
    off = pl.multiple_of(w * SLICE, 8)
    tbs, tsem = (tb0, tb1), (ts0, ts1)
    pltpu.sync_copy(tc_hbm.at[pl.ds(off, SLICE)], accs)

    def issue(v, p):
        pltpu.async_copy(
            ps_hbm.at[pl.ds(pl.multiple_of(v * BP + off, 8), SLICE)],
            tbs[p], tsem[p])

    issue(0, 0)

    def outer(vo, _):
        for p in (0, 1):
            v = 2 * vo + p

            @pl.when(v + 1 < NW)
            def _():
                issue(v + 1, 1 - p)

            pltpu.make_async_copy(
                ps_hbm.at[pl.ds(pl.multiple_of(v * BP + off, 8), SLICE)],
                tbs[p], tsem[p]).wait()

            tb = tbs[p]

            @plsc.parallel_loop(0, SLICE // L, unroll=4)
            def sub(k):
                sl = pl.ds(k * L, L)
                accs[sl] = accs[sl] - tb[sl]
        return 0

    lax.fori_loop(0, NW // 2, outer, 0)

    pltpu.sync_copy(na_hbm.at[pl.ds(off, SLICE)], tb0)

    @plsc.parallel_loop(0, SLICE // L, unroll=4)
    def div(k):
        sl = pl.ds(k * L, L)
        accs[sl] = accs[sl] / tb0[sl]
    pltpu.sync_copy(accs, lo_hbm.at[pl.ds(off, SLICE)])


def _k3_body(x_hbm, b_hbm, lo_hbm, out_hbm, lt, bb0, bb1, vb0, vb1, ob0, ob1,
             lsem, bs0, bs1, vs0, vs1, os0, os1):
    w = _wid()
    bbs, vbs, obs = (bb0, bb1), (vb0, vb1), (ob0, ob1)
    bsem, vsem, osem = (bs0, bs1), (vs0, vs1), (os0, os1)

    pltpu.async_copy(lo_hbm, lt, lsem)

    def base(t):
        return pl.multiple_of(w * C + t * T, 8)

    def issue(t, p):
        pltpu.async_copy(b_hbm.at[pl.ds(base(t), T)], bbs[p], bsem[p])
        pltpu.async_copy(x_hbm.at[pl.ds(base(t), T)], vbs[p], vsem[p])

    issue(0, 0)
    pltpu.make_async_copy(lo_hbm, lt, lsem).wait()

    def outer(to, _):
        for p in (0, 1):
            t = 2 * to + p

            @pl.when(t + 1 < NT)
            def _():
                issue(t + 1, 1 - p)

            pltpu.make_async_copy(b_hbm.at[pl.ds(base(t), T)], bbs[p],
                                  bsem[p]).wait()
            pltpu.make_async_copy(x_hbm.at[pl.ds(base(t), T)], vbs[p],
                                  vsem[p]).wait()

            @pl.when(t >= 2)
            def _():
                pltpu.make_async_copy(obs[p], out_hbm.at[pl.ds(base(t - 2), T)],
                                      osem[p]).wait()

            bb, vb, ob = bbs[p], vbs[p], obs[p]

            @plsc.parallel_loop(0, T // L, unroll=4)
            def vreg(k):
                sl = pl.ds(k * L, L)
                b = bb[sl]
                v = vb[sl]
                g = plsc.load_gather(lt, [b])
                ob[sl] = v + g
            pltpu.async_copy(obs[p], out_hbm.at[pl.ds(base(t), T)], osem[p])
        return 0

    lax.fori_loop(0, NT // 2, outer, 0)
    pltpu.make_async_copy(ob0, out_hbm.at[pl.ds(base(NT - 2), T)], os0).wait()
    pltpu.make_async_copy(ob1, out_hbm.at[pl.ds(base(NT - 1), T)], os1).wait()


_k1 = functools.partial(
    pl.kernel,
    out_type=jax.ShapeDtypeStruct((NW * BP,), jnp.float32),
    mesh=_MESH,
    compiler_params=_PARAMS,
    scratch_types=[
        pltpu.VMEM((BP,), jnp.float32),
        pltpu.VMEM((T + 2 * H,), jnp.int32),
        pltpu.VMEM((T + 2 * H,), jnp.int32),
        pltpu.VMEM((T,), jnp.float32),
        pltpu.VMEM((T,), jnp.float32),
        pltpu.SemaphoreType.DMA,
        pltpu.SemaphoreType.DMA,
        pltpu.SemaphoreType.DMA,
        pltpu.SemaphoreType.DMA,
    ],
)(_k1_body)

_k2 = functools.partial(
    pl.kernel,
    out_type=jax.ShapeDtypeStruct((BP,), jnp.float32),
    mesh=_MESH,
    compiler_params=_PARAMS,
    scratch_types=[
        pltpu.VMEM((SLICE,), jnp.float32),
        pltpu.VMEM((SLICE,), jnp.float32),
        pltpu.VMEM((SLICE,), jnp.float32),
        pltpu.SemaphoreType.DMA,
        pltpu.SemaphoreType.DMA,
    ],
)(_k2_body)

_k3 = functools.partial(
    pl.kernel,
    out_type=jax.ShapeDtypeStruct((N,), jnp.float32),
    mesh=_MESH,
    compiler_params=_PARAMS,
    scratch_types=[
        pltpu.VMEM((BP,), jnp.float32),
        pltpu.VMEM((T,), jnp.int32),
        pltpu.VMEM((T,), jnp.int32),
        pltpu.VMEM((T,), jnp.float32),
        pltpu.VMEM((T,), jnp.float32),
        pltpu.VMEM((T,), jnp.float32),
        pltpu.VMEM((T,), jnp.float32),
        pltpu.SemaphoreType.DMA,
        pltpu.SemaphoreType.DMA,
        pltpu.SemaphoreType.DMA,
        pltpu.SemaphoreType.DMA,
        pltpu.SemaphoreType.DMA,
        pltpu.SemaphoreType.DMA,
        pltpu.SemaphoreType.DMA,
    ],
)(_k3_body)


def kernel(node_outputs, batch, total_charge, n_atoms):
    x = node_outputs.reshape(N)
    tc = jnp.concatenate([total_charge, jnp.zeros((BP - B,), jnp.float32)])
    na = jnp.concatenate([n_atoms, jnp.ones((BP - B,), jnp.float32)])
    ps = _k1(x, batch)
    leftover = _k2(ps, tc, na)
    return _k3(x, batch, leftover)


# speedup vs baseline: 478.8558x; 1.0396x over previous
import functools

import jax
import jax.numpy as jnp
from jax import lax
from jax.experimental import pallas as pl
from jax.experimental.pallas import tpu as pltpu
from jax.experimental.pallas import tpu_sc as plsc

N = 6_400_000
B = 100_000
NC = 2
NS = 16
NW = NC * NS
C = N // NW
T = 4_000
NT = C // T
SLICE = 3_136
BP = NW * SLICE
L = 16
H = 16

_MESH = plsc.VectorSubcoreMesh(core_axis_name="c", subcore_axis_name="s")
_PARAMS = pltpu.CompilerParams(needs_layout_passes=False)


def _wid():
    return lax.axis_index("s") * NC + lax.axis_index("c")


def _k1_body(x_hbm, b_hbm, ps_hbm, acc, bb0, bb1, vb0, vb1, bs0, bs1, vs0, vs1):
    w = _wid()
    bbs, vbs = (bb0, bb1), (vb0, vb1)
    bsem, vsem = (bs0, bs1), (vs0, vs1)

    def base(t):
        return pl.multiple_of(w * C + t * T, 8)

    def issue(t, p):
        pltpu.async_copy(b_hbm.at[pl.ds(base(t), T)], bbs[p].at[pl.ds(H, T)],
                         bsem[p])
        pltpu.async_copy(x_hbm.at[pl.ds(base(t), T)], vbs[p], vsem[p])

    issue(0, 0)

    @plsc.parallel_loop(0, BP // L, unroll=8)
    def zero(i):
        acc[pl.ds(i * L, L)] = jnp.zeros((L,), jnp.float32)

    iota = lax.iota(jnp.int32, L)
    idx_r = jnp.maximum(iota - 1, 0)
    last = iota == (L - 1)
    first = iota == 0

    def compute(bb, vb):
        @plsc.parallel_loop(0, T // L, unroll=4)
        def vreg(k):
            off = H + k * L
            b = bb[pl.ds(off, L)]
            v = vb[pl.ds(k * L, L)]
            s = plsc.cumsum(v)
            b_next = bb[pl.ds(off + 1, L)]
            ei = (b != b_next).astype(jnp.int32)
            is_end = (ei != 0) | last
            is_start = (jnp.take_along_axis(ei, idx_r, axis=0) != 0) | first
            plsc.addupdate_scatter(acc, [b], s, mask=is_end)
            plsc.addupdate_scatter(acc, [b], v - s, mask=is_start)

    def outer(to, _):
        for p in (0, 1):
            t = 2 * to + p

            @pl.when(t + 1 < NT)
            def _():
                issue(t + 1, 1 - p)

            pltpu.make_async_copy(b_hbm.at[pl.ds(base(t), T)],
                                  bbs[p].at[pl.ds(H, T)], bsem[p]).wait()
            pltpu.make_async_copy(x_hbm.at[pl.ds(base(t), T)], vbs[p],
                                  vsem[p]).wait()
            compute(bbs[p], vbs[p])
        return 0

    lax.fori_loop(0, NT // 2, outer, 0)
    pltpu.sync_copy(acc, ps_hbm.at[pl.ds(pl.multiple_of(w * BP, 8), BP)])


def _k2_body(ps_hbm, tc_hbm, na_hbm, lo_hbm, accs, tb0, tb1, ts0, ts1):
    w = _wid()
    off = pl.multiple_of(w * SLICE, 8)
    tbs, tsem = (tb0, tb1), (ts0, ts1)
    pltpu.sync_copy(tc_hbm.at[pl.ds(off, SLICE)], accs)

    def issue(v, p):
        pltpu.async_copy(
            ps_hbm.at[pl.ds(pl.multiple_of(v * BP + off, 8), SLICE)],
            tbs[p], tsem[p])

    issue(0, 0)

    def outer(vo, _):
        for p in (0, 1):
            v = 2 * vo + p

            @pl.when(v + 1 < NW)
            def _():
                issue(v + 1, 1 - p)

            pltpu.make_async_copy(
                ps_hbm.at[pl.ds(pl.multiple_of(v * BP + off, 8), SLICE)],
                tbs[p], tsem[p]).wait()

            tb = tbs[p]

            @plsc.parallel_loop(0, SLICE // L, unroll=4)
            def sub(k):
                sl = pl.ds(k * L, L)
                accs[sl] = accs[sl] - tb[sl]
        return 0

    lax.fori_loop(0, NW // 2, outer, 0)

    pltpu.sync_copy(na_hbm.at[pl.ds(off, SLICE)], tb0)

    @plsc.parallel_loop(0, SLICE // L, unroll=4)
    def div(k):
        sl = pl.ds(k * L, L)
        accs[sl] = accs[sl] / tb0[sl]
    pltpu.sync_copy(accs, lo_hbm.at[pl.ds(off, SLICE)])


def _k3_body(x_hbm, b_hbm, lo_hbm, out_hbm, lt, bb0, bb1, vb0, vb1, ob0, ob1,
             lsem, bs0, bs1, vs0, vs1, os0, os1):
    w = _wid()
    bbs, vbs, obs = (bb0, bb1), (vb0, vb1), (ob0, ob1)
    bsem, vsem, osem = (bs0, bs1), (vs0, vs1), (os0, os1)

    pltpu.async_copy(lo_hbm, lt, lsem)

    def base(t):
        return pl.multiple_of(w * C + t * T, 8)

    def issue(t, p):
        pltpu.async_copy(b_hbm.at[pl.ds(base(t), T)], bbs[p], bsem[p])
        pltpu.async_copy(x_hbm.at[pl.ds(base(t), T)], vbs[p], vsem[p])

    issue(0, 0)
    pltpu.make_async_copy(lo_hbm, lt, lsem).wait()

    def outer(to, _):
        for p in (0, 1):
            t = 2 * to + p

            @pl.when(t + 1 < NT)
            def _():
                issue(t + 1, 1 - p)

            pltpu.make_async_copy(b_hbm.at[pl.ds(base(t), T)], bbs[p],
                                  bsem[p]).wait()
            pltpu.make_async_copy(x_hbm.at[pl.ds(base(t), T)], vbs[p],
                                  vsem[p]).wait()

            @pl.when(t >= 2)
            def _():
                pltpu.make_async_copy(obs[p], out_hbm.at[pl.ds(base(t - 2), T)],
                                      osem[p]).wait()

            bb, vb, ob = bbs[p], vbs[p], obs[p]

            @plsc.parallel_loop(0, T // L, unroll=4)
            def vreg(k):
                sl = pl.ds(k * L, L)
                b = bb[sl]
                v = vb[sl]
                g = plsc.load_gather(lt, [b])
                ob[sl] = v + g
            pltpu.async_copy(obs[p], out_hbm.at[pl.ds(base(t), T)], osem[p])
        return 0

    lax.fori_loop(0, NT // 2, outer, 0)
    pltpu.make_async_copy(ob0, out_hbm.at[pl.ds(base(NT - 2), T)], os0).wait()
    pltpu.make_async_copy(ob1, out_hbm.at[pl.ds(base(NT - 1), T)], os1).wait()


_k1 = functools.partial(
    pl.kernel,
    out_type=jax.ShapeDtypeStruct((NW * BP,), jnp.float32),
    mesh=_MESH,
    compiler_params=_PARAMS,
    scratch_types=[
        pltpu.VMEM((BP,), jnp.float32),
        pltpu.VMEM((T + 2 * H,), jnp.int32),
        pltpu.VMEM((T + 2 * H,), jnp.int32),
        pltpu.VMEM((T,), jnp.float32),
        pltpu.VMEM((T,), jnp.float32),
        pltpu.SemaphoreType.DMA,
        pltpu.SemaphoreType.DMA,
        pltpu.SemaphoreType.DMA,
        pltpu.SemaphoreType.DMA,
    ],
)(_k1_body)

_k2 = functools.partial(
    pl.kernel,
    out_type=jax.ShapeDtypeStruct((BP,), jnp.float32),
    mesh=_MESH,
    compiler_params=_PARAMS,
    scratch_types=[
        pltpu.VMEM((SLICE,), jnp.float32),
        pltpu.VMEM((SLICE,), jnp.float32),
        pltpu.VMEM((SLICE,), jnp.float32),
        pltpu.SemaphoreType.DMA,
        pltpu.SemaphoreType.DMA,
    ],
)(_k2_body)

_k3 = functools.partial(
    pl.kernel,
    out_type=jax.ShapeDtypeStruct((N,), jnp.float32),
    mesh=_MESH,
    compiler_params=_PARAMS,
    scratch_types=[
        pltpu.VMEM((BP,), jnp.float32),
        pltpu.VMEM((T,), jnp.int32),
        pltpu.VMEM((T,), jnp.int32),
        pltpu.VMEM((T,), jnp.float32),
        pltpu.VMEM((T,), jnp.float32),
        pltpu.VMEM((T,), jnp.float32),
        pltpu.VMEM((T,), jnp.float32),
        pltpu.SemaphoreType.DMA,
        pltpu.SemaphoreType.DMA,
        pltpu.SemaphoreType.DMA,
        pltpu.SemaphoreType.DMA,
        pltpu.SemaphoreType.DMA,
        pltpu.SemaphoreType.DMA,
        pltpu.SemaphoreType.DMA,
    ],
)(_k3_body)


def kernel(node_outputs, batch, total_charge, n_atoms):
    x = node_outputs.reshape(N)
    tc = jnp.concatenate([total_charge, jnp.zeros((BP - B,), jnp.float32)])
    na = jnp.concatenate([n_atoms, jnp.ones((BP - B,), jnp.float32)])
    ps = _k1(x, batch)
    leftover = _k2(ps, tc, na)
    return _k3(x, batch, leftover)


# speedup vs baseline: 584.6987x; 1.2210x over previous
import functools

import jax
import jax.numpy as jnp
from jax import lax
from jax.experimental import pallas as pl
from jax.experimental.pallas import tpu as pltpu
from jax.experimental.pallas import tpu_sc as plsc

N = 6_400_000
B = 100_000
NC = 2
NS = 16
NW = NC * NS
C = N // NW
T = 4_000
NT = C // T
SLICE = 3_136
BP = NW * SLICE
L = 16
H = 16
IMAX = 2147483647

_MESH = plsc.VectorSubcoreMesh(core_axis_name="c", subcore_axis_name="s")
_PARAMS = pltpu.CompilerParams(needs_layout_passes=False)


def _wid():
    return lax.axis_index("s") * NC + lax.axis_index("c")


def _lane(vec, i, iota):
    return jnp.min(jnp.where(iota == i, vec, IMAX))


def _k1_body(x_hbm, b_hbm, ps_hbm, rng_hbm, acc, bb0, bb1, vb0, vb1, bnd,
             rngbuf, bs0, bs1, vs0, vs1, bndsem):
    w = _wid()
    bbs, vbs = (bb0, bb1), (vb0, vb1)
    bsem, vsem = (bs0, bs1), (vs0, vs1)

    def base(t):
        return pl.multiple_of(w * C + t * T, 8)

    def issue(t, p):
        pltpu.async_copy(b_hbm.at[pl.ds(base(t), T)], bbs[p].at[pl.ds(H, T)],
                         bsem[p])
        pltpu.async_copy(x_hbm.at[pl.ds(base(t), T)], vbs[p], vsem[p])

    first_at = pl.multiple_of(w * C, 8)
    last_at = pl.multiple_of(w * C + C - L, 8)
    pltpu.async_copy(b_hbm.at[pl.ds(first_at, L)], bnd.at[pl.ds(0, L)], bndsem)
    pltpu.async_copy(b_hbm.at[pl.ds(last_at, L)], bnd.at[pl.ds(L, L)], bndsem)
    issue(0, 0)

    iota = lax.iota(jnp.int32, L)
    idx_r = jnp.maximum(iota - 1, 0)
    last = iota == (L - 1)
    first = iota == 0

    pltpu.make_async_copy(b_hbm.at[pl.ds(first_at, L)], bnd.at[pl.ds(0, L)],
                          bndsem).wait()
    pltpu.make_async_copy(b_hbm.at[pl.ds(last_at, L)], bnd.at[pl.ds(L, L)],
                          bndsem).wait()
    blk_lo = _lane(bnd[pl.ds(0, L)], 0, iota) // SLICE
    blk_hi = _lane(bnd[pl.ds(L, L)], L - 1, iota) // SLICE

    def zblk(j, _):
        off0 = pl.multiple_of(j * SLICE, 8)

        @plsc.parallel_loop(0, SLICE // L, unroll=8)
        def zero(i):
            acc[pl.ds(off0 + i * L, L)] = jnp.zeros((L,), jnp.float32)

        return 0

    lax.fori_loop(blk_lo, blk_hi + 1, zblk, 0)

    def compute(bb, vb):
        @plsc.parallel_loop(0, T // L, unroll=4)
        def vreg(k):
            off = H + k * L
            b = bb[pl.ds(off, L)]
            v = vb[pl.ds(k * L, L)]
            s = plsc.cumsum(v)
            b_next = bb[pl.ds(off + 1, L)]
            ei = (b != b_next).astype(jnp.int32)
            is_end = (ei != 0) | last
            is_start = (jnp.take_along_axis(ei, idx_r, axis=0) != 0) | first
            plsc.addupdate_scatter(acc, [b], s, mask=is_end)
            plsc.addupdate_scatter(acc, [b], v - s, mask=is_start)

    def outer(to, _):
        for p in (0, 1):
            t = 2 * to + p

            @pl.when(t + 1 < NT)
            def _():
                issue(t + 1, 1 - p)

            pltpu.make_async_copy(b_hbm.at[pl.ds(base(t), T)],
                                  bbs[p].at[pl.ds(H, T)], bsem[p]).wait()
            pltpu.make_async_copy(x_hbm.at[pl.ds(base(t), T)], vbs[p],
                                  vsem[p]).wait()
            compute(bbs[p], vbs[p])
        return 0

    lax.fori_loop(0, NT // 2, outer, 0)

    def wblk(j, _):
        off0 = pl.multiple_of(j * SLICE, 8)
        pltpu.sync_copy(
            acc.at[pl.ds(off0, SLICE)],
            ps_hbm.at[pl.ds(pl.multiple_of(w * BP + j * SLICE, 8), SLICE)])
        return 0

    lax.fori_loop(blk_lo, blk_hi + 1, wblk, 0)

    rngbuf[pl.ds(0, L)] = jnp.where(iota == 0, blk_lo,
                                    jnp.where(iota == 1, blk_hi, 0))
    pltpu.sync_copy(rngbuf, rng_hbm.at[pl.ds(pl.multiple_of(w * L, 8), L)])


def _k2_body(ps_hbm, tc_hbm, na_hbm, rng_hbm, lo_hbm, accs, tbuf, rngb):
    w = _wid()
    off = pl.multiple_of(w * SLICE, 8)
    iota = lax.iota(jnp.int32, L)
    pltpu.sync_copy(rng_hbm, rngb)
    pltpu.sync_copy(tc_hbm.at[pl.ds(off, SLICE)], accs)

    def row(v, _):
        rec = rngb[pl.ds(pl.multiple_of(v * L, 8), L)]
        blk_lo = _lane(rec, 0, iota)
        blk_hi = _lane(rec, 1, iota)

        @pl.when((blk_lo <= w) & (w <= blk_hi))
        def _():
            pltpu.sync_copy(
                ps_hbm.at[pl.ds(pl.multiple_of(v * BP + w * SLICE, 8), SLICE)],
                tbuf)

            @plsc.parallel_loop(0, SLICE // L, unroll=4)
            def sub(k):
                sl = pl.ds(k * L, L)
                accs[sl] = accs[sl] - tbuf[sl]

        return 0

    lax.fori_loop(0, NW, row, 0)

    pltpu.sync_copy(na_hbm.at[pl.ds(off, SLICE)], tbuf)

    @plsc.parallel_loop(0, SLICE // L, unroll=4)
    def div(k):
        sl = pl.ds(k * L, L)
        accs[sl] = accs[sl] / tbuf[sl]

    pltpu.sync_copy(accs, lo_hbm.at[pl.ds(off, SLICE)])


def _k3_body(x_hbm, b_hbm, lo_hbm, out_hbm, lt, bb0, bb1, vb0, vb1, ob0, ob1,
             bnd, bs0, bs1, vs0, vs1, os0, os1, bndsem):
    w = _wid()
    bbs, vbs, obs = (bb0, bb1), (vb0, vb1), (ob0, ob1)
    bsem, vsem, osem = (bs0, bs1), (vs0, vs1), (os0, os1)

    def base(t):
        return pl.multiple_of(w * C + t * T, 8)

    def issue(t, p):
        pltpu.async_copy(b_hbm.at[pl.ds(base(t), T)], bbs[p], bsem[p])
        pltpu.async_copy(x_hbm.at[pl.ds(base(t), T)], vbs[p], vsem[p])

    first_at = pl.multiple_of(w * C, 8)
    last_at = pl.multiple_of(w * C + C - L, 8)
    pltpu.async_copy(b_hbm.at[pl.ds(first_at, L)], bnd.at[pl.ds(0, L)], bndsem)
    pltpu.async_copy(b_hbm.at[pl.ds(last_at, L)], bnd.at[pl.ds(L, L)], bndsem)
    issue(0, 0)

    iota = lax.iota(jnp.int32, L)
    pltpu.make_async_copy(b_hbm.at[pl.ds(first_at, L)], bnd.at[pl.ds(0, L)],
                          bndsem).wait()
    pltpu.make_async_copy(b_hbm.at[pl.ds(last_at, L)], bnd.at[pl.ds(L, L)],
                          bndsem).wait()
    blk_lo = _lane(bnd[pl.ds(0, L)], 0, iota) // SLICE
    blk_hi = _lane(bnd[pl.ds(L, L)], L - 1, iota) // SLICE

    def lblk(j, _):
        off0 = pl.multiple_of(j * SLICE, 8)
        pltpu.sync_copy(lo_hbm.at[pl.ds(off0, SLICE)],
                        lt.at[pl.ds(off0, SLICE)])
        return 0

    lax.fori_loop(blk_lo, blk_hi + 1, lblk, 0)

    def outer(to, _):
        for p in (0, 1):
            t = 2 * to + p

            @pl.when(t + 1 < NT)
            def _():
                issue(t + 1, 1 - p)

            pltpu.make_async_copy(b_hbm.at[pl.ds(base(t), T)], bbs[p],
                                  bsem[p]).wait()
            pltpu.make_async_copy(x_hbm.at[pl.ds(base(t), T)], vbs[p],
                                  vsem[p]).wait()

            @pl.when(t >= 2)
            def _():
                pltpu.make_async_copy(obs[p], out_hbm.at[pl.ds(base(t - 2), T)],
                                      osem[p]).wait()

            bb, vb, ob = bbs[p], vbs[p], obs[p]

            @plsc.parallel_loop(0, T // L, unroll=4)
            def vreg(k):
                sl = pl.ds(k * L, L)
                b = bb[sl]
                v = vb[sl]
                g = plsc.load_gather(lt, [b])
                ob[sl] = v + g

            pltpu.async_copy(obs[p], out_hbm.at[pl.ds(base(t), T)], osem[p])
        return 0

    lax.fori_loop(0, NT // 2, outer, 0)
    pltpu.make_async_copy(ob0, out_hbm.at[pl.ds(base(NT - 2), T)], os0).wait()
    pltpu.make_async_copy(ob1, out_hbm.at[pl.ds(base(NT - 1), T)], os1).wait()


_k1 = functools.partial(
    pl.kernel,
    out_type=(jax.ShapeDtypeStruct((NW * BP,), jnp.float32),
              jax.ShapeDtypeStruct((NW * L,), jnp.int32)),
    mesh=_MESH,
    compiler_params=_PARAMS,
    scratch_types=[
        pltpu.VMEM((BP,), jnp.float32),
        pltpu.VMEM((T + 2 * H,), jnp.int32),
        pltpu.VMEM((T + 2 * H,), jnp.int32),
        pltpu.VMEM((T,), jnp.float32),
        pltpu.VMEM((T,), jnp.float32),
        pltpu.VMEM((2 * L,), jnp.int32),
        pltpu.VMEM((L,), jnp.int32),
        pltpu.SemaphoreType.DMA,
        pltpu.SemaphoreType.DMA,
        pltpu.SemaphoreType.DMA,
        pltpu.SemaphoreType.DMA,
        pltpu.SemaphoreType.DMA,
    ],
)(_k1_body)

_k2 = functools.partial(
    pl.kernel,
    out_type=jax.ShapeDtypeStruct((BP,), jnp.float32),
    mesh=_MESH,
    compiler_params=_PARAMS,
    scratch_types=[
        pltpu.VMEM((SLICE,), jnp.float32),
        pltpu.VMEM((SLICE,), jnp.float32),
        pltpu.VMEM((NW * L,), jnp.int32),
    ],
)(_k2_body)

_k3 = functools.partial(
    pl.kernel,
    out_type=jax.ShapeDtypeStruct((N,), jnp.float32),
    mesh=_MESH,
    compiler_params=_PARAMS,
    scratch_types=[
        pltpu.VMEM((BP,), jnp.float32),
        pltpu.VMEM((T,), jnp.int32),
        pltpu.VMEM((T,), jnp.int32),
        pltpu.VMEM((T,), jnp.float32),
        pltpu.VMEM((T,), jnp.float32),
        pltpu.VMEM((T,), jnp.float32),
        pltpu.VMEM((T,), jnp.float32),
        pltpu.VMEM((2 * L,), jnp.int32),
        pltpu.SemaphoreType.DMA,
        pltpu.SemaphoreType.DMA,
        pltpu.SemaphoreType.DMA,
        pltpu.SemaphoreType.DMA,
        pltpu.SemaphoreType.DMA,
        pltpu.SemaphoreType.DMA,
        pltpu.SemaphoreType.DMA,
    ],
)(_k3_body)


def kernel(node_outputs, batch, total_charge, n_atoms):
    x = node_outputs.reshape(N)
    tc = jnp.concatenate([total_charge, jnp.zeros((BP - B,), jnp.float32)])
    na = jnp.concatenate([n_atoms, jnp.ones((BP - B,), jnp.float32)])
    ps, rng = _k1(x, batch)
    leftover = _k2(ps, tc, na, rng)
    return _k3(x, batch, leftover)


# speedup vs baseline: 590.5363x; 1.0100x over previous
import functools

import jax
import jax.numpy as jnp
from jax import lax
from jax.experimental import pallas as pl
from jax.experimental.pallas import tpu as pltpu
from jax.experimental.pallas import tpu_sc as plsc

N = 6_400_000
B = 100_000
NC = 2
NS = 16
NW = NC * NS
C = N // NW
T = 4_000
NT = C // T
SLICE = 3_136
BP = NW * SLICE
L = 16
H = 16
IMAX = 2147483647

_MESH = plsc.VectorSubcoreMesh(core_axis_name="c", subcore_axis_name="s")
_PARAMS = pltpu.CompilerParams(needs_layout_passes=False)


def _wid():
    return lax.axis_index("s") * NC + lax.axis_index("c")


def _lane(vec, i, iota):
    return jnp.min(jnp.where(iota == i, vec, IMAX))


def _k1_body(x_hbm, b_hbm, ps_hbm, rng_hbm, acc, bb0, bb1, vb0, vb1, bnd,
             rngbuf, bs0, bs1, vs0, vs1, bndsem):
    w = _wid()
    bbs, vbs = (bb0, bb1), (vb0, vb1)
    bsem, vsem = (bs0, bs1), (vs0, vs1)

    def base(t):
        return pl.multiple_of(w * C + t * T, 8)

    def issue(t, p):
        pltpu.async_copy(b_hbm.at[pl.ds(base(t), T)], bbs[p].at[pl.ds(H, T)],
                         bsem[p])
        pltpu.async_copy(x_hbm.at[pl.ds(base(t), T)], vbs[p], vsem[p])

    first_at = pl.multiple_of(w * C, 8)
    last_at = pl.multiple_of(w * C + C - L, 8)
    pltpu.async_copy(b_hbm.at[pl.ds(first_at, L)], bnd.at[pl.ds(0, L)], bndsem)
    pltpu.async_copy(b_hbm.at[pl.ds(last_at, L)], bnd.at[pl.ds(L, L)], bndsem)
    issue(0, 0)

    iota = lax.iota(jnp.int32, L)
    idx_r = jnp.maximum(iota - 1, 0)
    last = iota == (L - 1)
    first = iota == 0

    pltpu.make_async_copy(b_hbm.at[pl.ds(first_at, L)], bnd.at[pl.ds(0, L)],
                          bndsem).wait()
    pltpu.make_async_copy(b_hbm.at[pl.ds(last_at, L)], bnd.at[pl.ds(L, L)],
                          bndsem).wait()
    blk_lo = _lane(bnd[pl.ds(0, L)], 0, iota) // SLICE
    blk_hi = _lane(bnd[pl.ds(L, L)], L - 1, iota) // SLICE

    def zblk(j, _):
        off0 = pl.multiple_of(j * SLICE, 8)

        @plsc.parallel_loop(0, SLICE // L, unroll=8)
        def zero(i):
            acc[pl.ds(off0 + i * L, L)] = jnp.zeros((L,), jnp.float32)

        return 0

    lax.fori_loop(blk_lo, blk_hi + 1, zblk, 0)

    def compute(bb, vb):
        @plsc.parallel_loop(0, T // L, unroll=4)
        def vreg(k):
            off = H + k * L
            b = bb[pl.ds(off, L)]
            v = vb[pl.ds(k * L, L)]
            s = plsc.cumsum(v)
            b_next = bb[pl.ds(off + 1, L)]
            ei = (b != b_next).astype(jnp.int32)
            is_end = (ei != 0) | last
            is_start = (jnp.take_along_axis(ei, idx_r, axis=0) != 0) | first
            plsc.addupdate_scatter(acc, [b], s, mask=is_end)
            plsc.addupdate_scatter(acc, [b], v - s, mask=is_start)

    def outer(to, _):
        for p in (0, 1):
            t = 2 * to + p

            @pl.when(t + 1 < NT)
            def _():
                issue(t + 1, 1 - p)

            pltpu.make_async_copy(b_hbm.at[pl.ds(base(t), T)],
                                  bbs[p].at[pl.ds(H, T)], bsem[p]).wait()
            pltpu.make_async_copy(x_hbm.at[pl.ds(base(t), T)], vbs[p],
                                  vsem[p]).wait()
            compute(bbs[p], vbs[p])
        return 0

    lax.fori_loop(0, NT // 2, outer, 0)

    def wblk(j, _):
        off0 = pl.multiple_of(j * SLICE, 8)
        pltpu.sync_copy(
            acc.at[pl.ds(off0, SLICE)],
            ps_hbm.at[pl.ds(pl.multiple_of(w * BP + j * SLICE, 8), SLICE)])
        return 0

    lax.fori_loop(blk_lo, blk_hi + 1, wblk, 0)

    rngbuf[pl.ds(0, L)] = jnp.where(iota == 0, blk_lo,
                                    jnp.where(iota == 1, blk_hi, 0))
    pltpu.sync_copy(rngbuf, rng_hbm.at[pl.ds(pl.multiple_of(w * L, 8), L)])


def _k3_body(x_hbm, b_hbm, ps_hbm, tc_hbm, na_hbm, rng_hbm, out_hbm, lt, bb0,
             bb1, vb0, vb1, ob0, ob1, bnd, tbuf, rngb, bs0, bs1, vs0, vs1, os0,
             os1, bndsem):
    w = _wid()
    bbs, vbs, obs = (bb0, bb1), (vb0, vb1), (ob0, ob1)
    bsem, vsem, osem = (bs0, bs1), (vs0, vs1), (os0, os1)

    def base(t):
        return pl.multiple_of(w * C + t * T, 8)

    def issue(t, p):
        pltpu.async_copy(b_hbm.at[pl.ds(base(t), T)], bbs[p], bsem[p])
        pltpu.async_copy(x_hbm.at[pl.ds(base(t), T)], vbs[p], vsem[p])

    first_at = pl.multiple_of(w * C, 8)
    last_at = pl.multiple_of(w * C + C - L, 8)
    pltpu.async_copy(b_hbm.at[pl.ds(first_at, L)], bnd.at[pl.ds(0, L)], bndsem)
    pltpu.async_copy(b_hbm.at[pl.ds(last_at, L)], bnd.at[pl.ds(L, L)], bndsem)
    issue(0, 0)

    iota = lax.iota(jnp.int32, L)
    pltpu.make_async_copy(b_hbm.at[pl.ds(first_at, L)], bnd.at[pl.ds(0, L)],
                          bndsem).wait()
    pltpu.make_async_copy(b_hbm.at[pl.ds(last_at, L)], bnd.at[pl.ds(L, L)],
                          bndsem).wait()
    blk_lo = _lane(bnd[pl.ds(0, L)], 0, iota) // SLICE
    blk_hi = _lane(bnd[pl.ds(L, L)], L - 1, iota) // SLICE

    pltpu.sync_copy(rng_hbm, rngb)

    def blkinit(j, _):
        off0 = pl.multiple_of(j * SLICE, 8)
        pltpu.sync_copy(tc_hbm.at[pl.ds(off0, SLICE)], lt.at[pl.ds(off0, SLICE)])

        def row(v, __):
            rec = rngb[pl.ds(pl.multiple_of(v * L, 8), L)]
            lo_v = _lane(rec, 0, iota)
            hi_v = _lane(rec, 1, iota)

            @pl.when((lo_v <= j) & (j <= hi_v))
            def _():
                pltpu.sync_copy(
                    ps_hbm.at[pl.ds(pl.multiple_of(v * BP + j * SLICE, 8),
                                    SLICE)], tbuf)

                @plsc.parallel_loop(0, SLICE // L, unroll=4)
                def sub(k):
                    sl2 = pl.ds(off0 + k * L, L)
                    lt[sl2] = lt[sl2] - tbuf[pl.ds(k * L, L)]

            return 0

        lax.fori_loop(0, NW, row, 0)
        pltpu.sync_copy(na_hbm.at[pl.ds(off0, SLICE)], tbuf)

        @plsc.parallel_loop(0, SLICE // L, unroll=4)
        def dv(k):
            sl2 = pl.ds(off0 + k * L, L)
            lt[sl2] = lt[sl2] / tbuf[pl.ds(k * L, L)]

        return 0

    lax.fori_loop(blk_lo, blk_hi + 1, blkinit, 0)

    def outer(to, _):
        for p in (0, 1):
            t = 2 * to + p

            @pl.when(t + 1 < NT)
            def _():
                issue(t + 1, 1 - p)

            pltpu.make_async_copy(b_hbm.at[pl.ds(base(t), T)], bbs[p],
                                  bsem[p]).wait()
            pltpu.make_async_copy(x_hbm.at[pl.ds(base(t), T)], vbs[p],
                                  vsem[p]).wait()

            @pl.when(t >= 2)
            def _():
                pltpu.make_async_copy(obs[p], out_hbm.at[pl.ds(base(t - 2), T)],
                                      osem[p]).wait()

            bb, vb, ob = bbs[p], vbs[p], obs[p]

            @plsc.parallel_loop(0, T // L, unroll=4)
            def vreg(k):
                sl = pl.ds(k * L, L)
                b = bb[sl]
                v = vb[sl]
                g = plsc.load_gather(lt, [b])
                ob[sl] = v + g

            pltpu.async_copy(obs[p], out_hbm.at[pl.ds(base(t), T)], osem[p])
        return 0

    lax.fori_loop(0, NT // 2, outer, 0)
    pltpu.make_async_copy(ob0, out_hbm.at[pl.ds(base(NT - 2), T)], os0).wait()
    pltpu.make_async_copy(ob1, out_hbm.at[pl.ds(base(NT - 1), T)], os1).wait()


_k1 = functools.partial(
    pl.kernel,
    out_type=(jax.ShapeDtypeStruct((NW * BP,), jnp.float32),
              jax.ShapeDtypeStruct((NW * L,), jnp.int32)),
    mesh=_MESH,
    compiler_params=_PARAMS,
    scratch_types=[
        pltpu.VMEM((BP,), jnp.float32),
        pltpu.VMEM((T + 2 * H,), jnp.int32),
        pltpu.VMEM((T + 2 * H,), jnp.int32),
        pltpu.VMEM((T,), jnp.float32),
        pltpu.VMEM((T,), jnp.float32),
        pltpu.VMEM((2 * L,), jnp.int32),
        pltpu.VMEM((L,), jnp.int32),
        pltpu.SemaphoreType.DMA,
        pltpu.SemaphoreType.DMA,
        pltpu.SemaphoreType.DMA,
        pltpu.SemaphoreType.DMA,
        pltpu.SemaphoreType.DMA,
    ],
)(_k1_body)

_k3 = functools.partial(
    pl.kernel,
    out_type=jax.ShapeDtypeStruct((N,), jnp.float32),
    mesh=_MESH,
    compiler_params=_PARAMS,
    scratch_types=[
        pltpu.VMEM((BP,), jnp.float32),
        pltpu.VMEM((T,), jnp.int32),
        pltpu.VMEM((T,), jnp.int32),
        pltpu.VMEM((T,), jnp.float32),
        pltpu.VMEM((T,), jnp.float32),
        pltpu.VMEM((T,), jnp.float32),
        pltpu.VMEM((T,), jnp.float32),
        pltpu.VMEM((2 * L,), jnp.int32),
        pltpu.VMEM((SLICE,), jnp.float32),
        pltpu.VMEM((NW * L,), jnp.int32),
        pltpu.SemaphoreType.DMA,
        pltpu.SemaphoreType.DMA,
        pltpu.SemaphoreType.DMA,
        pltpu.SemaphoreType.DMA,
        pltpu.SemaphoreType.DMA,
        pltpu.SemaphoreType.DMA,
        pltpu.SemaphoreType.DMA,
    ],
)(_k3_body)


def kernel(node_outputs, batch, total_charge, n_atoms):
    x = node_outputs.reshape(N)
    tc = jnp.concatenate([total_charge, jnp.zeros((BP - B,), jnp.float32)])
    na = jnp.concatenate([n_atoms, jnp.ones((BP - B,), jnp.float32)])
    ps, rng = _k1(x, batch)
    return _k3(x, batch, ps, tc, na, rng)
